# baseline (device time: 918755 ns/iter reference)
import jax
import jax.numpy as jnp
from jax import lax
from jax.experimental import pallas as pl
from jax.experimental.pallas import tpu as pltpu

N_DEV = 4
F8 = jnp.float8_e4m3fn


def _snap_e4m3(a):
    u = lax.bitcast_convert_type(a, jnp.uint32)
    ur = (u + jnp.uint32(0x7FFFF) + ((u >> 20) & jnp.uint32(1))) & jnp.uint32(
        0xFFF00000)
    an = lax.bitcast_convert_type(ur, jnp.float32)
    magic = jnp.float32(12582912.0)
    asub = ((a * jnp.float32(512.0) + magic) - magic) * jnp.float32(1.0 / 512.0)
    s = jnp.where(a >= jnp.float32(2.0 ** -6), an, asub)
    return jnp.minimum(s, jnp.float32(448.0))


def _fused_ar_epilogue(partial):
    M, N = partial.shape
    C = M // N_DEV
    H = N // 2
    TILE = 128
    p4 = partial.reshape(N_DEV, C, N)

    def body(p_ref, out_q, out_y, amax_out, rsA, rsB,
             acc_a, acc_b, pa, pb, va, vb, qa, qb,
             qa_t, qb_t, ya_t, yb_t, ax_send, ax_slots,
             rsA_ss, rsA_rs, rsB_ss, rsB_rs,
             agA_ss, agA_rs, agB_ss, agB_rs,
             ax_ss, ax_rs, lsem_a, lsem_b):
        d = lax.axis_index("i")
        rA = lax.rem(d + 1, N_DEV)
        rB = lax.rem(d + 3, N_DEV)

        colA = pl.ds(0, H)
        colB = pl.ds(H, H)

        cpa = pltpu.make_async_copy(p_ref.at[d, :, colA], acc_a, lsem_a)
        cpb = pltpu.make_async_copy(p_ref.at[d, :, colB], acc_b, lsem_b)
        cpa.start()
        cpb.start()
        cpa.wait()
        cpb.wait()

        SB = C // 2

        def rs_desc(ring, s, b):
            acc, scr, ss, rs_, dev = (
                (acc_a, rsA, rsA_ss, rsA_rs, rA) if ring == 0
                else (acc_b, rsB, rsB_ss, rsB_rs, rB))
            rows = pl.ds(b * SB, SB)
            return pltpu.make_async_remote_copy(
                src_ref=acc.at[rows, :], dst_ref=scr.at[s, rows, :],
                send_sem=ss.at[2 * s + b], recv_sem=rs_.at[2 * s + b],
                device_id=(dev,), device_id_type=pl.DeviceIdType.MESH)

        am = jnp.float32(0.0)
        for b in range(2):
            rs_desc(0, 0, b).start()
            rs_desc(1, 0, b).start()
        for s in range(N_DEV - 1):
            cA = lax.rem(d - 1 - s + 2 * N_DEV, N_DEV)
            cB = lax.rem(d + 1 + s, N_DEV)
            last = s == N_DEV - 2
            for b in range(2):
                rs_desc(0, s, b).wait()
                rs_desc(1, s, b).wait()
                for j in range(SB // TILE):
                    rows = pl.ds(b * SB + j * TILE, TILE)
                    c1 = pltpu.make_async_copy(
                        p_ref.at[cA, rows, colA], pa, lsem_a)
                    c2 = pltpu.make_async_copy(
                        rsA.at[s, rows, :], va, lsem_a)
                    c3 = pltpu.make_async_copy(
                        p_ref.at[cB, rows, colB], pb, lsem_b)
                    c4 = pltpu.make_async_copy(
                        rsB.at[s, rows, :], vb, lsem_b)
                    c1.start()
                    c3.start()
                    c2.start()
                    c4.start()
                    c1.wait()
                    c2.wait()
                    ta = pa[:, :] + va[:, :]
                    if last:
                        ta = jnp.maximum(ta, jnp.float32(0.0))
                        am = jnp.maximum(am, jnp.max(ta))
                    acc_a[rows, :] = ta
                    c3.wait()
                    c4.wait()
                    tb = pb[:, :] + vb[:, :]
                    if last:
                        tb = jnp.maximum(tb, jnp.float32(0.0))
                        am = jnp.maximum(am, jnp.max(tb))
                    acc_b[rows, :] = tb
                if not last:
                    rs_desc(0, s + 1, b).start()
                    rs_desc(1, s + 1, b).start()

        gA = lax.rem(d + 1, N_DEV)
        gB = lax.rem(d + 3, N_DEV)

        ax_send[...] = jnp.zeros((8, 128), jnp.float32) + am

        sends = []
        for k in range(1, N_DEV):
            tgt = lax.rem(d + k, N_DEV)
            rd = pltpu.make_async_remote_copy(
                src_ref=ax_send, dst_ref=ax_slots.at[d],
                send_sem=ax_ss.at[k - 1], recv_sem=ax_rs.at[d],
                device_id=(tgt,), device_id_type=pl.DeviceIdType.MESH)
            rd.start()
            sends.append(rd)
        cp = pltpu.make_async_copy(ax_send, ax_slots.at[d], lsem_a)
        cp.start()
        cp.wait()
        for k in range(1, N_DEV):
            src = lax.rem(d - k + 2 * N_DEV, N_DEV)
            rd = pltpu.make_async_remote_copy(
                src_ref=ax_send, dst_ref=ax_slots.at[src],
                send_sem=ax_ss.at[k - 1], recv_sem=ax_rs.at[src],
                device_id=(src,), device_id_type=pl.DeviceIdType.MESH)
            rd.wait_recv()
        for rd in sends:
            rd.wait_send()
        gmax = jnp.max(ax_slots[...])
        amax_out[...] = jnp.zeros((8, 128), jnp.float32) + gmax
        scale = gmax / jnp.float32(448.0)

        inv = jnp.float32(1.0) / scale
        for j in range(C // TILE):
            rows = pl.ds(j * TILE, TILE)
            qa[rows, :] = _snap_e4m3(acc_a[rows, :] * inv).astype(F8)
            qb[rows, :] = _snap_e4m3(acc_b[rows, :] * inv).astype(F8)
        cpa = pltpu.make_async_copy(qa, out_q.at[gA, :, colA], lsem_a)
        cpb = pltpu.make_async_copy(qb, out_q.at[gB, :, colB], lsem_b)
        cpa.start()
        cpb.start()
        cpa.wait()
        cpb.wait()

        def deq_hbm(chunk, col, q8t, yt, lsem):
            for j in range(C // TILE):
                rows = pl.ds(j * TILE, TILE)
                cq = pltpu.make_async_copy(out_q.at[chunk, rows, col], q8t, lsem)
                cq.start()
                cq.wait()
                yt[...] = (q8t[...].astype(jnp.float32) * scale).astype(
                    jnp.bfloat16)
                cy = pltpu.make_async_copy(yt, out_y.at[chunk, rows, col], lsem)
                cy.start()
                cy.wait()

        def deq_own(qsrc, chunk, col, yt, lsem):
            for j in range(C // TILE):
                rows = pl.ds(j * TILE, TILE)
                yt[...] = (qsrc[rows, :].astype(jnp.float32) * scale).astype(
                    jnp.bfloat16)
                cy = pltpu.make_async_copy(yt, out_y.at[chunk, rows, col], lsem)
                cy.start()
                cy.wait()

        for t in range(N_DEV - 1):
            sA = lax.rem(d + 1 - t + 2 * N_DEV, N_DEV)
            sB = lax.rem(d + 3 + t, N_DEV)
            srcA = qa if t == 0 else out_q.at[
                lax.rem(d - t + 1 + 2 * N_DEV, N_DEV), :, colA]
            srcB = qb if t == 0 else out_q.at[
                lax.rem(d + t - 1 + 2 * N_DEV, N_DEV), :, colB]
            rdA = pltpu.make_async_remote_copy(
                src_ref=srcA, dst_ref=out_q.at[sA, :, colA],
                send_sem=agA_ss.at[t], recv_sem=agA_rs.at[t],
                device_id=(rA,), device_id_type=pl.DeviceIdType.MESH)
            rdB = pltpu.make_async_remote_copy(
                src_ref=srcB, dst_ref=out_q.at[sB, :, colB],
                send_sem=agB_ss.at[t], recv_sem=agB_rs.at[t],
                device_id=(rB,), device_id_type=pl.DeviceIdType.MESH)
            rdA.start()
            rdB.start()
            if t == 0:
                deq_own(qa, gA, colA, ya_t, lsem_a)
                deq_own(qb, gB, colB, yb_t, lsem_b)
            else:
                deq_hbm(lax.rem(d - t + 1 + 2 * N_DEV, N_DEV), colA,
                        qa_t, ya_t, lsem_a)
                deq_hbm(lax.rem(d + t - 1 + 2 * N_DEV, N_DEV), colB,
                        qb_t, yb_t, lsem_b)
            rdA.wait()
            rdB.wait()
        deq_hbm(lax.rem(d - 2 + 2 * N_DEV, N_DEV), colA, qa_t, ya_t, lsem_a)
        deq_hbm(lax.rem(d + 2, N_DEV), colB, qb_t, yb_t, lsem_b)

    out_q, out_y, amax_out, _, _ = pl.pallas_call(
        body,
        out_shape=[
            jax.ShapeDtypeStruct((N_DEV, C, N), F8),
            jax.ShapeDtypeStruct((N_DEV, C, N), jnp.bfloat16),
            jax.ShapeDtypeStruct((8, 128), jnp.float32),
            jax.ShapeDtypeStruct((N_DEV - 1, C, H), jnp.float32),
            jax.ShapeDtypeStruct((N_DEV - 1, C, H), jnp.float32),
        ],
        in_specs=[pl.BlockSpec(memory_space=pltpu.HBM)],
        out_specs=[
            pl.BlockSpec(memory_space=pltpu.HBM),
            pl.BlockSpec(memory_space=pltpu.HBM),
            pl.BlockSpec(memory_space=pltpu.VMEM),
            pl.BlockSpec(memory_space=pltpu.HBM),
            pl.BlockSpec(memory_space=pltpu.HBM),
        ],
        scratch_shapes=[
            pltpu.VMEM((C, H), jnp.float32),
            pltpu.VMEM((C, H), jnp.float32),
            pltpu.VMEM((TILE, H), jnp.float32),
            pltpu.VMEM((TILE, H), jnp.float32),
            pltpu.VMEM((TILE, H), jnp.float32),
            pltpu.VMEM((TILE, H), jnp.float32),
            pltpu.VMEM((C, H), F8),
            pltpu.VMEM((C, H), F8),
            pltpu.VMEM((TILE, H), F8),
            pltpu.VMEM((TILE, H), F8),
            pltpu.VMEM((TILE, H), jnp.bfloat16),
            pltpu.VMEM((TILE, H), jnp.bfloat16),
            pltpu.VMEM((8, 128), jnp.float32),
            pltpu.VMEM((N_DEV, 8, 128), jnp.float32),
            pltpu.SemaphoreType.DMA((2 * (N_DEV - 1),)),
            pltpu.SemaphoreType.DMA((2 * (N_DEV - 1),)),
            pltpu.SemaphoreType.DMA((2 * (N_DEV - 1),)),
            pltpu.SemaphoreType.DMA((2 * (N_DEV - 1),)),
            pltpu.SemaphoreType.DMA((N_DEV - 1,)),
            pltpu.SemaphoreType.DMA((N_DEV - 1,)),
            pltpu.SemaphoreType.DMA((N_DEV - 1,)),
            pltpu.SemaphoreType.DMA((N_DEV - 1,)),
            pltpu.SemaphoreType.DMA((N_DEV - 1,)),
            pltpu.SemaphoreType.DMA((N_DEV,)),
            pltpu.SemaphoreType.DMA,
            pltpu.SemaphoreType.DMA,
        ],
        compiler_params=pltpu.CompilerParams(
            vmem_limit_bytes=63 * 1024 * 1024,
        ),
    )(p4)
    return out_y


def kernel(x, w_mat):
    partial = jnp.dot(x, w_mat, preferred_element_type=jnp.float32)
    y = _fused_ar_epilogue(partial)
    return y.reshape(partial.shape)


# device time: 902898 ns/iter; 1.0176x vs baseline; 1.0176x over previous
import jax
import jax.numpy as jnp
from jax import lax
from jax.experimental import pallas as pl
from jax.experimental.pallas import tpu as pltpu

N_DEV = 4
F8 = jnp.float8_e4m3fn


def _snap_e4m3(a):
    u = lax.bitcast_convert_type(a, jnp.uint32)
    ur = (u + jnp.uint32(0x7FFFF) + ((u >> 20) & jnp.uint32(1))) & jnp.uint32(
        0xFFF00000)
    an = lax.bitcast_convert_type(ur, jnp.float32)
    magic = jnp.float32(12582912.0)
    asub = ((a * jnp.float32(512.0) + magic) - magic) * jnp.float32(1.0 / 512.0)
    s = jnp.where(a >= jnp.float32(2.0 ** -6), an, asub)
    return jnp.minimum(s, jnp.float32(448.0))


def _fused_ar_epilogue(partial):
    M, N = partial.shape
    C = M // N_DEV
    H = N // 2
    TILE = 128
    p4 = partial.reshape(N_DEV, C, N)

    def body(p_ref, out_q, out_y, amax_out, rsA, rsB,
             acc_a, acc_b, pa, pb, va, vb, qa, qb,
             qa_t, ya_t, yb_t, ax_send, ax_slots,
             rsA_ss, rsA_rs, rsB_ss, rsB_rs,
             agA_ss, agA_rs, agB_ss, agB_rs,
             ax_ss, ax_rs, lsem_a, lsem_b):
        d = lax.axis_index("i")
        rA = lax.rem(d + 1, N_DEV)
        rB = lax.rem(d + 3, N_DEV)

        colA = pl.ds(0, H)
        colB = pl.ds(H, H)

        cpa = pltpu.make_async_copy(p_ref.at[d, :, colA], acc_a, lsem_a)
        cpb = pltpu.make_async_copy(p_ref.at[d, :, colB], acc_b, lsem_b)
        cpa.start()
        cpb.start()
        cpa.wait()
        cpb.wait()

        SB = C // 2

        def rs_desc(ring, s, b):
            acc, scr, ss, rs_, dev = (
                (acc_a, rsA, rsA_ss, rsA_rs, rA) if ring == 0
                else (acc_b, rsB, rsB_ss, rsB_rs, rB))
            rows = pl.ds(b * SB, SB)
            return pltpu.make_async_remote_copy(
                src_ref=acc.at[rows, :], dst_ref=scr.at[s, rows, :],
                send_sem=ss.at[2 * s + b], recv_sem=rs_.at[2 * s + b],
                device_id=(dev,), device_id_type=pl.DeviceIdType.MESH)

        am = jnp.float32(0.0)
        for b in range(2):
            rs_desc(0, 0, b).start()
            rs_desc(1, 0, b).start()
        for s in range(N_DEV - 1):
            cA = lax.rem(d - 1 - s + 2 * N_DEV, N_DEV)
            cB = lax.rem(d + 1 + s, N_DEV)
            last = s == N_DEV - 2
            for b in range(2):
                rs_desc(0, s, b).wait()
                rs_desc(1, s, b).wait()
                for j in range(SB // TILE):
                    rows = pl.ds(b * SB + j * TILE, TILE)
                    c1 = pltpu.make_async_copy(
                        p_ref.at[cA, rows, colA], pa, lsem_a)
                    c2 = pltpu.make_async_copy(
                        rsA.at[s, rows, :], va, lsem_a)
                    c3 = pltpu.make_async_copy(
                        p_ref.at[cB, rows, colB], pb, lsem_b)
                    c4 = pltpu.make_async_copy(
                        rsB.at[s, rows, :], vb, lsem_b)
                    c1.start()
                    c3.start()
                    c2.start()
                    c4.start()
                    c1.wait()
                    c2.wait()
                    ta = pa[:, :] + va[:, :]
                    if last:
                        ta = jnp.maximum(ta, jnp.float32(0.0))
                        am = jnp.maximum(am, jnp.max(ta))
                    acc_a[rows, :] = ta
                    c3.wait()
                    c4.wait()
                    tb = pb[:, :] + vb[:, :]
                    if last:
                        tb = jnp.maximum(tb, jnp.float32(0.0))
                        am = jnp.maximum(am, jnp.max(tb))
                    acc_b[rows, :] = tb
                if not last:
                    rs_desc(0, s + 1, b).start()
                    rs_desc(1, s + 1, b).start()

        gA = lax.rem(d + 1, N_DEV)
        gB = lax.rem(d + 3, N_DEV)

        ax_send[...] = jnp.zeros((8, 128), jnp.float32) + am

        sends = []
        for k in range(1, N_DEV):
            tgt = lax.rem(d + k, N_DEV)
            rd = pltpu.make_async_remote_copy(
                src_ref=ax_send, dst_ref=ax_slots.at[d],
                send_sem=ax_ss.at[k - 1], recv_sem=ax_rs.at[d],
                device_id=(tgt,), device_id_type=pl.DeviceIdType.MESH)
            rd.start()
            sends.append(rd)
        cp = pltpu.make_async_copy(ax_send, ax_slots.at[d], lsem_a)
        cp.start()
        cp.wait()
        for k in range(1, N_DEV):
            src = lax.rem(d - k + 2 * N_DEV, N_DEV)
            rd = pltpu.make_async_remote_copy(
                src_ref=ax_send, dst_ref=ax_slots.at[src],
                send_sem=ax_ss.at[k - 1], recv_sem=ax_rs.at[src],
                device_id=(src,), device_id_type=pl.DeviceIdType.MESH)
            rd.wait_recv()
        for rd in sends:
            rd.wait_send()
        gmax = jnp.max(ax_slots[...])
        amax_out[...] = jnp.zeros((8, 128), jnp.float32) + gmax
        scale = gmax / jnp.float32(448.0)

        inv = jnp.float32(1.0) / scale
        for j in range(C // TILE):
            rows = pl.ds(j * TILE, TILE)
            qa[rows, :] = _snap_e4m3(acc_a[rows, :] * inv).astype(F8)
            qb[rows, :] = _snap_e4m3(acc_b[rows, :] * inv).astype(F8)
        cpa = pltpu.make_async_copy(qa, out_q.at[gA, :, colA], lsem_a)
        cpb = pltpu.make_async_copy(qb, out_q.at[gB, :, colB], lsem_b)
        cpa.start()
        cpb.start()
        cpa.wait()
        cpb.wait()

        DTILE = 256

        def deq_own(qsrc, chunk, col, yt, lsem):
            for j in range(C // DTILE):
                rows = pl.ds(j * DTILE, DTILE)
                yt[...] = (qsrc[rows, :].astype(jnp.float32) * scale).astype(
                    jnp.bfloat16)
                cy = pltpu.make_async_copy(yt, out_y.at[chunk, rows, col], lsem)
                cy.start()
                cy.wait()

        def deq_hbm(chunk, col, qbuf, yt, lsem):
            cq = pltpu.make_async_copy(out_q.at[chunk, :, col], qbuf, lsem)
            cq.start()
            cq.wait()
            deq_own(qbuf, chunk, col, yt, lsem)

        for t in range(N_DEV - 1):
            sA = lax.rem(d + 1 - t + 2 * N_DEV, N_DEV)
            sB = lax.rem(d + 3 + t, N_DEV)
            srcA = qa if t == 0 else out_q.at[
                lax.rem(d - t + 1 + 2 * N_DEV, N_DEV), :, colA]
            srcB = qb if t == 0 else out_q.at[
                lax.rem(d + t - 1 + 2 * N_DEV, N_DEV), :, colB]
            rdA = pltpu.make_async_remote_copy(
                src_ref=srcA, dst_ref=out_q.at[sA, :, colA],
                send_sem=agA_ss.at[t], recv_sem=agA_rs.at[t],
                device_id=(rA,), device_id_type=pl.DeviceIdType.MESH)
            rdB = pltpu.make_async_remote_copy(
                src_ref=srcB, dst_ref=out_q.at[sB, :, colB],
                send_sem=agB_ss.at[t], recv_sem=agB_rs.at[t],
                device_id=(rB,), device_id_type=pl.DeviceIdType.MESH)
            rdA.start()
            rdB.start()
            if t == 0:
                deq_own(qa, gA, colA, ya_t, lsem_a)
                deq_own(qb, gB, colB, yb_t, lsem_b)
            else:
                deq_hbm(lax.rem(d - t + 1 + 2 * N_DEV, N_DEV), colA,
                        qa_t, ya_t, lsem_a)
                deq_hbm(lax.rem(d + t - 1 + 2 * N_DEV, N_DEV), colB,
                        qa_t, yb_t, lsem_b)
            rdA.wait()
            rdB.wait()
        deq_hbm(lax.rem(d - 2 + 2 * N_DEV, N_DEV), colA, qa_t, ya_t, lsem_a)
        deq_hbm(lax.rem(d + 2, N_DEV), colB, qa_t, yb_t, lsem_b)

    out_q, out_y, amax_out, _, _ = pl.pallas_call(
        body,
        out_shape=[
            jax.ShapeDtypeStruct((N_DEV, C, N), F8),
            jax.ShapeDtypeStruct((N_DEV, C, N), jnp.bfloat16),
            jax.ShapeDtypeStruct((8, 128), jnp.float32),
            jax.ShapeDtypeStruct((N_DEV - 1, C, H), jnp.float32),
            jax.ShapeDtypeStruct((N_DEV - 1, C, H), jnp.float32),
        ],
        in_specs=[pl.BlockSpec(memory_space=pltpu.HBM)],
        out_specs=[
            pl.BlockSpec(memory_space=pltpu.HBM),
            pl.BlockSpec(memory_space=pltpu.HBM),
            pl.BlockSpec(memory_space=pltpu.VMEM),
            pl.BlockSpec(memory_space=pltpu.HBM),
            pl.BlockSpec(memory_space=pltpu.HBM),
        ],
        scratch_shapes=[
            pltpu.VMEM((C, H), jnp.float32),
            pltpu.VMEM((C, H), jnp.float32),
            pltpu.VMEM((TILE, H), jnp.float32),
            pltpu.VMEM((TILE, H), jnp.float32),
            pltpu.VMEM((TILE, H), jnp.float32),
            pltpu.VMEM((TILE, H), jnp.float32),
            pltpu.VMEM((C, H), F8),
            pltpu.VMEM((C, H), F8),
            pltpu.VMEM((C, H), F8),
            pltpu.VMEM((256, H), jnp.bfloat16),
            pltpu.VMEM((256, H), jnp.bfloat16),
            pltpu.VMEM((8, 128), jnp.float32),
            pltpu.VMEM((N_DEV, 8, 128), jnp.float32),
            pltpu.SemaphoreType.DMA((2 * (N_DEV - 1),)),
            pltpu.SemaphoreType.DMA((2 * (N_DEV - 1),)),
            pltpu.SemaphoreType.DMA((2 * (N_DEV - 1),)),
            pltpu.SemaphoreType.DMA((2 * (N_DEV - 1),)),
            pltpu.SemaphoreType.DMA((N_DEV - 1,)),
            pltpu.SemaphoreType.DMA((N_DEV - 1,)),
            pltpu.SemaphoreType.DMA((N_DEV - 1,)),
            pltpu.SemaphoreType.DMA((N_DEV - 1,)),
            pltpu.SemaphoreType.DMA((N_DEV - 1,)),
            pltpu.SemaphoreType.DMA((N_DEV,)),
            pltpu.SemaphoreType.DMA,
            pltpu.SemaphoreType.DMA,
        ],
        compiler_params=pltpu.CompilerParams(
            vmem_limit_bytes=63 * 1024 * 1024,
        ),
    )(p4)
    return out_y


def kernel(x, w_mat):
    partial = jnp.dot(x, w_mat, preferred_element_type=jnp.float32)
    y = _fused_ar_epilogue(partial)
    return y.reshape(partial.shape)


# device time: 810268 ns/iter; 1.1339x vs baseline; 1.1143x over previous
import jax
import jax.numpy as jnp
from jax import lax
from jax.experimental import pallas as pl
from jax.experimental.pallas import tpu as pltpu

N_DEV = 4
F8 = jnp.float8_e4m3fn


def _snap_e4m3(a):
    u = lax.bitcast_convert_type(a, jnp.uint32)
    ur = (u + jnp.uint32(0x7FFFF) + ((u >> 20) & jnp.uint32(1))) & jnp.uint32(
        0xFFF00000)
    an = lax.bitcast_convert_type(ur, jnp.float32)
    magic = jnp.float32(12582912.0)
    asub = ((a * jnp.float32(512.0) + magic) - magic) * jnp.float32(1.0 / 512.0)
    s = jnp.where(a >= jnp.float32(2.0 ** -6), an, asub)
    return jnp.minimum(s, jnp.float32(448.0))


def _fused_gemm_ar_epilogue(x4, w):
    K = x4.shape[2]
    C = x4.shape[1]
    N = w.shape[1]
    H = N // 2
    TILE = 128
    SB = C // 2

    def body(x_ref, w_ref, out_q, amax_out, rsA, rsB,
             acc_a, acc_b, wv, xa, xb, va, vb, q8a, q8b, ax_send, ax_slots,
             rsA_ss, rsA_rs, rsB_ss, rsB_rs,
             agA_ss, agA_rs, agB_ss, agB_rs,
             ax_ss, ax_rs, lsem_a, lsem_b):
        d = lax.axis_index("i")
        rA = lax.rem(d + 1, N_DEV)
        rB = lax.rem(d + 3, N_DEV)

        colA = pl.ds(0, H)
        colB = pl.ds(H, H)

        def rs_desc(ring, s, b):
            acc, scr, ss, rs_, dev = (
                (acc_a, rsA, rsA_ss, rsA_rs, rA) if ring == 0
                else (acc_b, rsB, rsB_ss, rsB_rs, rB))
            rows = pl.ds(b * SB, SB)
            return pltpu.make_async_remote_copy(
                src_ref=acc.at[rows, :], dst_ref=scr.at[s, rows, :],
                send_sem=ss.at[2 * s + b], recv_sem=rs_.at[2 * s + b],
                device_id=(dev,), device_id_type=pl.DeviceIdType.MESH)

        cw = pltpu.make_async_copy(w_ref, wv, lsem_a)
        cx = pltpu.make_async_copy(x_ref.at[d], xa, lsem_b)
        cw.start()
        cx.start()
        cw.wait()
        cx.wait()
        def seed_body(j, carry):
            rows = pl.ds(j * TILE, TILE)
            xt = xa[rows, :]
            acc_a[rows, :] = jnp.dot(
                xt, wv[:, 0:H], preferred_element_type=jnp.float32)
            acc_b[rows, :] = jnp.dot(
                xt, wv[:, H:2 * H], preferred_element_type=jnp.float32)
            return carry

        for b in range(2):
            lax.fori_loop(b * (SB // TILE), (b + 1) * (SB // TILE),
                          seed_body, 0)
            rs_desc(0, 0, b).start()
            rs_desc(1, 0, b).start()

        am = jnp.float32(0.0)
        for s in range(N_DEV - 1):
            cA = lax.rem(d - 1 - s + 2 * N_DEV, N_DEV)
            cB = lax.rem(d + 1 + s, N_DEV)
            cxa = pltpu.make_async_copy(x_ref.at[cA], xa, lsem_a)
            cxb = pltpu.make_async_copy(x_ref.at[cB], xb, lsem_b)
            cxa.start()
            cxb.start()
            cxa.wait()
            cxb.wait()
            last = s == N_DEV - 2

            def accum_body(j, am_c, s=s, last=last):
                rows = pl.ds(j * TILE, TILE)
                c2 = pltpu.make_async_copy(rsA.at[s, rows, :], va, lsem_a)
                c4 = pltpu.make_async_copy(rsB.at[s, rows, :], vb, lsem_b)
                c2.start()
                c4.start()
                c2.wait()
                ta = jnp.dot(xa[rows, :], wv[:, 0:H],
                             preferred_element_type=jnp.float32) + va[:, :]
                if last:
                    ta = jnp.maximum(ta, jnp.float32(0.0))
                    am_c = jnp.maximum(am_c, jnp.max(ta))
                acc_a[rows, :] = ta
                c4.wait()
                tb = jnp.dot(xb[rows, :], wv[:, H:2 * H],
                             preferred_element_type=jnp.float32) + vb[:, :]
                if last:
                    tb = jnp.maximum(tb, jnp.float32(0.0))
                    am_c = jnp.maximum(am_c, jnp.max(tb))
                acc_b[rows, :] = tb
                return am_c

            for b in range(2):
                rs_desc(0, s, b).wait()
                rs_desc(1, s, b).wait()
                am = lax.fori_loop(b * (SB // TILE), (b + 1) * (SB // TILE),
                                   accum_body, am)
                if not last:
                    rs_desc(0, s + 1, b).start()
                    rs_desc(1, s + 1, b).start()

        gA = lax.rem(d + 1, N_DEV)
        gB = lax.rem(d + 3, N_DEV)

        ax_send[...] = jnp.zeros((8, 128), jnp.float32) + am

        sends = []
        for k in range(1, N_DEV):
            tgt = lax.rem(d + k, N_DEV)
            rd = pltpu.make_async_remote_copy(
                src_ref=ax_send, dst_ref=ax_slots.at[d],
                send_sem=ax_ss.at[k - 1], recv_sem=ax_rs.at[d],
                device_id=(tgt,), device_id_type=pl.DeviceIdType.MESH)
            rd.start()
            sends.append(rd)
        cp = pltpu.make_async_copy(ax_send, ax_slots.at[d], lsem_a)
        cp.start()
        cp.wait()
        for k in range(1, N_DEV):
            src = lax.rem(d - k + 2 * N_DEV, N_DEV)
            rd = pltpu.make_async_remote_copy(
                src_ref=ax_send, dst_ref=ax_slots.at[src],
                send_sem=ax_ss.at[k - 1], recv_sem=ax_rs.at[src],
                device_id=(src,), device_id_type=pl.DeviceIdType.MESH)
            rd.wait_recv()
        for rd in sends:
            rd.wait_send()
        gmax = jnp.max(ax_slots[...])
        amax_out[...] = jnp.zeros((8, 128), jnp.float32) + gmax
        scale = gmax / jnp.float32(448.0)

        inv = jnp.float32(1.0) / scale

        def snap_body(j, carry):
            rows = pl.ds(j * TILE, TILE)
            q8a[...] = _snap_e4m3(acc_a[rows, :] * inv).astype(F8)
            ca = pltpu.make_async_copy(q8a, out_q.at[gA, rows, colA], lsem_a)
            ca.start()
            q8b[...] = _snap_e4m3(acc_b[rows, :] * inv).astype(F8)
            cb = pltpu.make_async_copy(q8b, out_q.at[gB, rows, colB], lsem_b)
            cb.start()
            ca.wait()
            cb.wait()
            return carry

        lax.fori_loop(0, C // TILE, snap_body, 0)

        for t in range(N_DEV - 1):
            sA = lax.rem(d + 1 - t + 2 * N_DEV, N_DEV)
            sB = lax.rem(d + 3 + t, N_DEV)
            rdA = pltpu.make_async_remote_copy(
                src_ref=out_q.at[sA, :, colA], dst_ref=out_q.at[sA, :, colA],
                send_sem=agA_ss.at[t], recv_sem=agA_rs.at[t],
                device_id=(rA,), device_id_type=pl.DeviceIdType.MESH)
            rdB = pltpu.make_async_remote_copy(
                src_ref=out_q.at[sB, :, colB], dst_ref=out_q.at[sB, :, colB],
                send_sem=agB_ss.at[t], recv_sem=agB_rs.at[t],
                device_id=(rB,), device_id_type=pl.DeviceIdType.MESH)
            rdA.start()
            rdB.start()
            rdA.wait()
            rdB.wait()

    out_q, amax_out, _, _ = pl.pallas_call(
        body,
        out_shape=[
            jax.ShapeDtypeStruct((N_DEV, C, N), F8),
            jax.ShapeDtypeStruct((8, 128), jnp.float32),
            jax.ShapeDtypeStruct((N_DEV - 1, C, H), jnp.float32),
            jax.ShapeDtypeStruct((N_DEV - 1, C, H), jnp.float32),
        ],
        in_specs=[
            pl.BlockSpec(memory_space=pltpu.HBM),
            pl.BlockSpec(memory_space=pltpu.HBM),
        ],
        out_specs=[
            pl.BlockSpec(memory_space=pltpu.HBM),
            pl.BlockSpec(memory_space=pltpu.VMEM),
            pl.BlockSpec(memory_space=pltpu.HBM),
            pl.BlockSpec(memory_space=pltpu.HBM),
        ],
        scratch_shapes=[
            pltpu.VMEM((C, H), jnp.float32),
            pltpu.VMEM((C, H), jnp.float32),
            pltpu.VMEM((K, N), jnp.bfloat16),
            pltpu.VMEM((C, K), jnp.bfloat16),
            pltpu.VMEM((C, K), jnp.bfloat16),
            pltpu.VMEM((TILE, H), jnp.float32),
            pltpu.VMEM((TILE, H), jnp.float32),
            pltpu.VMEM((TILE, H), F8),
            pltpu.VMEM((TILE, H), F8),
            pltpu.VMEM((8, 128), jnp.float32),
            pltpu.VMEM((N_DEV, 8, 128), jnp.float32),
            pltpu.SemaphoreType.DMA((2 * (N_DEV - 1),)),
            pltpu.SemaphoreType.DMA((2 * (N_DEV - 1),)),
            pltpu.SemaphoreType.DMA((2 * (N_DEV - 1),)),
            pltpu.SemaphoreType.DMA((2 * (N_DEV - 1),)),
            pltpu.SemaphoreType.DMA((N_DEV - 1,)),
            pltpu.SemaphoreType.DMA((N_DEV - 1,)),
            pltpu.SemaphoreType.DMA((N_DEV - 1,)),
            pltpu.SemaphoreType.DMA((N_DEV - 1,)),
            pltpu.SemaphoreType.DMA((N_DEV - 1,)),
            pltpu.SemaphoreType.DMA((N_DEV,)),
            pltpu.SemaphoreType.DMA,
            pltpu.SemaphoreType.DMA,
        ],
        compiler_params=pltpu.CompilerParams(
            vmem_limit_bytes=63 * 1024 * 1024,
        ),
    )(x4, w)
    return out_q, amax_out


def kernel(x, w_mat):
    M, K = x.shape
    N = w_mat.shape[1]
    x4 = x.astype(jnp.bfloat16).reshape(N_DEV, M // N_DEV, K)
    q, amax = _fused_gemm_ar_epilogue(x4, w_mat.astype(jnp.bfloat16))
    scale = amax[0, 0] / jnp.float32(448.0)
    y = q.reshape(M, N).astype(jnp.float32) * scale
    return y.astype(jnp.bfloat16)


# device time: 796225 ns/iter; 1.1539x vs baseline; 1.0176x over previous
import jax
import jax.numpy as jnp
from jax import lax
from jax.experimental import pallas as pl
from jax.experimental.pallas import tpu as pltpu

N_DEV = 4
F8 = jnp.float8_e4m3fn


def _snap_e4m3(a):
    u = lax.bitcast_convert_type(a, jnp.uint32)
    ur = (u + jnp.uint32(0x7FFFF) + ((u >> 20) & jnp.uint32(1))) & jnp.uint32(
        0xFFF00000)
    an = lax.bitcast_convert_type(ur, jnp.float32)
    magic = jnp.float32(12582912.0)
    asub = ((a * jnp.float32(512.0) + magic) - magic) * jnp.float32(1.0 / 512.0)
    s = jnp.where(a >= jnp.float32(2.0 ** -6), an, asub)
    return jnp.minimum(s, jnp.float32(448.0))


def _fused_gemm_ar_epilogue(x4, w):
    K = x4.shape[2]
    C = x4.shape[1]
    N = w.shape[1]
    H = N // 2
    TILE = 128
    SB = C // 4
    NSB = 4

    def body(x_ref, w_ref, out_q, amax_out, rsA, rsB,
             acc_a, acc_b, wv, xa, xb, va, vb, q8a, q8b, ax_send, ax_slots,
             rsA_ss, rsA_rs, rsB_ss, rsB_rs,
             agA_ss, agA_rs, agB_ss, agB_rs,
             ax_ss, ax_rs, lsem_a, lsem_b):
        d = lax.axis_index("i")
        rA = lax.rem(d + 1, N_DEV)
        rB = lax.rem(d + 3, N_DEV)

        colA = pl.ds(0, H)
        colB = pl.ds(H, H)

        def rs_desc(ring, s, b):
            acc, scr, ss, rs_, dev = (
                (acc_a, rsA, rsA_ss, rsA_rs, rA) if ring == 0
                else (acc_b, rsB, rsB_ss, rsB_rs, rB))
            rows = pl.ds(b * SB, SB)
            return pltpu.make_async_remote_copy(
                src_ref=acc.at[rows, :], dst_ref=scr.at[s, rows, :],
                send_sem=ss.at[NSB * s + b], recv_sem=rs_.at[NSB * s + b],
                device_id=(dev,), device_id_type=pl.DeviceIdType.MESH)

        cw = pltpu.make_async_copy(w_ref, wv, lsem_a)
        cx = pltpu.make_async_copy(x_ref.at[d], xa, lsem_b)
        cw.start()
        cx.start()
        cw.wait()
        cx.wait()
        def seed_body(j, carry):
            rows = pl.ds(j * TILE, TILE)
            xt = xa[rows, :]
            acc_a[rows, :] = jnp.dot(
                xt, wv[:, 0:H], preferred_element_type=jnp.float32)
            acc_b[rows, :] = jnp.dot(
                xt, wv[:, H:2 * H], preferred_element_type=jnp.float32)
            return carry

        for b in range(NSB):
            lax.fori_loop(b * (SB // TILE), (b + 1) * (SB // TILE),
                          seed_body, 0)
            rs_desc(0, 0, b).start()
            rs_desc(1, 0, b).start()

        am = jnp.float32(0.0)
        for s in range(N_DEV - 1):
            cA = lax.rem(d - 1 - s + 2 * N_DEV, N_DEV)
            cB = lax.rem(d + 1 + s, N_DEV)
            cxa = pltpu.make_async_copy(x_ref.at[cA], xa, lsem_a)
            cxb = pltpu.make_async_copy(x_ref.at[cB], xb, lsem_b)
            cxa.start()
            cxb.start()
            cxa.wait()
            cxb.wait()
            last = s == N_DEV - 2

            def accum_body(j, am_c, s=s, last=last):
                rows = pl.ds(j * TILE, TILE)
                c2 = pltpu.make_async_copy(rsA.at[s, rows, :], va, lsem_a)
                c4 = pltpu.make_async_copy(rsB.at[s, rows, :], vb, lsem_b)
                c2.start()
                c4.start()
                c2.wait()
                ta = jnp.dot(xa[rows, :], wv[:, 0:H],
                             preferred_element_type=jnp.float32) + va[:, :]
                if last:
                    ta = jnp.maximum(ta, jnp.float32(0.0))
                    am_c = jnp.maximum(am_c, jnp.max(ta))
                acc_a[rows, :] = ta
                c4.wait()
                tb = jnp.dot(xb[rows, :], wv[:, H:2 * H],
                             preferred_element_type=jnp.float32) + vb[:, :]
                if last:
                    tb = jnp.maximum(tb, jnp.float32(0.0))
                    am_c = jnp.maximum(am_c, jnp.max(tb))
                acc_b[rows, :] = tb
                return am_c

            for b in range(NSB):
                rs_desc(0, s, b).wait()
                rs_desc(1, s, b).wait()
                am = lax.fori_loop(b * (SB // TILE), (b + 1) * (SB // TILE),
                                   accum_body, am)
                if not last:
                    rs_desc(0, s + 1, b).start()
                    rs_desc(1, s + 1, b).start()

        gA = lax.rem(d + 1, N_DEV)
        gB = lax.rem(d + 3, N_DEV)

        ax_send[...] = jnp.zeros((8, 128), jnp.float32) + am

        sends = []
        for k in range(1, N_DEV):
            tgt = lax.rem(d + k, N_DEV)
            rd = pltpu.make_async_remote_copy(
                src_ref=ax_send, dst_ref=ax_slots.at[d],
                send_sem=ax_ss.at[k - 1], recv_sem=ax_rs.at[d],
                device_id=(tgt,), device_id_type=pl.DeviceIdType.MESH)
            rd.start()
            sends.append(rd)
        cp = pltpu.make_async_copy(ax_send, ax_slots.at[d], lsem_a)
        cp.start()
        cp.wait()
        for k in range(1, N_DEV):
            src = lax.rem(d - k + 2 * N_DEV, N_DEV)
            rd = pltpu.make_async_remote_copy(
                src_ref=ax_send, dst_ref=ax_slots.at[src],
                send_sem=ax_ss.at[k - 1], recv_sem=ax_rs.at[src],
                device_id=(src,), device_id_type=pl.DeviceIdType.MESH)
            rd.wait_recv()
        for rd in sends:
            rd.wait_send()
        gmax = jnp.max(ax_slots[...])
        amax_out[...] = jnp.zeros((8, 128), jnp.float32) + gmax
        scale = gmax / jnp.float32(448.0)

        inv = jnp.float32(1.0) / scale

        def snap_body(j, carry):
            rows = pl.ds(j * TILE, TILE)
            q8a[...] = _snap_e4m3(acc_a[rows, :] * inv).astype(F8)
            ca = pltpu.make_async_copy(q8a, out_q.at[gA, rows, colA], lsem_a)
            ca.start()
            q8b[...] = _snap_e4m3(acc_b[rows, :] * inv).astype(F8)
            cb = pltpu.make_async_copy(q8b, out_q.at[gB, rows, colB], lsem_b)
            cb.start()
            ca.wait()
            cb.wait()
            return carry

        lax.fori_loop(0, C // TILE, snap_body, 0)

        for t in range(N_DEV - 1):
            sA = lax.rem(d + 1 - t + 2 * N_DEV, N_DEV)
            sB = lax.rem(d + 3 + t, N_DEV)
            rdA = pltpu.make_async_remote_copy(
                src_ref=out_q.at[sA, :, colA], dst_ref=out_q.at[sA, :, colA],
                send_sem=agA_ss.at[t], recv_sem=agA_rs.at[t],
                device_id=(rA,), device_id_type=pl.DeviceIdType.MESH)
            rdB = pltpu.make_async_remote_copy(
                src_ref=out_q.at[sB, :, colB], dst_ref=out_q.at[sB, :, colB],
                send_sem=agB_ss.at[t], recv_sem=agB_rs.at[t],
                device_id=(rB,), device_id_type=pl.DeviceIdType.MESH)
            rdA.start()
            rdB.start()
            rdA.wait()
            rdB.wait()

    out_q, amax_out, _, _ = pl.pallas_call(
        body,
        out_shape=[
            jax.ShapeDtypeStruct((N_DEV, C, N), F8),
            jax.ShapeDtypeStruct((8, 128), jnp.float32),
            jax.ShapeDtypeStruct((N_DEV - 1, C, H), jnp.float32),
            jax.ShapeDtypeStruct((N_DEV - 1, C, H), jnp.float32),
        ],
        in_specs=[
            pl.BlockSpec(memory_space=pltpu.HBM),
            pl.BlockSpec(memory_space=pltpu.HBM),
        ],
        out_specs=[
            pl.BlockSpec(memory_space=pltpu.HBM),
            pl.BlockSpec(memory_space=pltpu.VMEM),
            pl.BlockSpec(memory_space=pltpu.HBM),
            pl.BlockSpec(memory_space=pltpu.HBM),
        ],
        scratch_shapes=[
            pltpu.VMEM((C, H), jnp.float32),
            pltpu.VMEM((C, H), jnp.float32),
            pltpu.VMEM((K, N), jnp.bfloat16),
            pltpu.VMEM((C, K), jnp.bfloat16),
            pltpu.VMEM((C, K), jnp.bfloat16),
            pltpu.VMEM((TILE, H), jnp.float32),
            pltpu.VMEM((TILE, H), jnp.float32),
            pltpu.VMEM((TILE, H), F8),
            pltpu.VMEM((TILE, H), F8),
            pltpu.VMEM((8, 128), jnp.float32),
            pltpu.VMEM((N_DEV, 8, 128), jnp.float32),
            pltpu.SemaphoreType.DMA((4 * (N_DEV - 1),)),
            pltpu.SemaphoreType.DMA((4 * (N_DEV - 1),)),
            pltpu.SemaphoreType.DMA((4 * (N_DEV - 1),)),
            pltpu.SemaphoreType.DMA((4 * (N_DEV - 1),)),
            pltpu.SemaphoreType.DMA((N_DEV - 1,)),
            pltpu.SemaphoreType.DMA((N_DEV - 1,)),
            pltpu.SemaphoreType.DMA((N_DEV - 1,)),
            pltpu.SemaphoreType.DMA((N_DEV - 1,)),
            pltpu.SemaphoreType.DMA((N_DEV - 1,)),
            pltpu.SemaphoreType.DMA((N_DEV,)),
            pltpu.SemaphoreType.DMA,
            pltpu.SemaphoreType.DMA,
        ],
        compiler_params=pltpu.CompilerParams(
            vmem_limit_bytes=63 * 1024 * 1024,
        ),
    )(x4, w)
    return out_q, amax_out


def kernel(x, w_mat):
    M, K = x.shape
    N = w_mat.shape[1]
    x4 = x.astype(jnp.bfloat16).reshape(N_DEV, M // N_DEV, K)
    q, amax = _fused_gemm_ar_epilogue(x4, w_mat.astype(jnp.bfloat16))
    scale = amax[0, 0] / jnp.float32(448.0)
    y = q.reshape(M, N).astype(jnp.float32) * scale
    return y.astype(jnp.bfloat16)
